# baseline (speedup 1.0000x reference)
import functools

import jax
import jax.numpy as jnp
from jax import lax
from jax.experimental import pallas as pl
from jax.experimental.pallas import tpu as pltpu
from jax.experimental.pallas import tpu_sc as plsc

D_VOCAB = 50257
D_MODEL = 1024
NC = 2
NS = 16
NW = NC * NS

PER_W = 512
SIZES = [40] * 12 + [32]
OFFS = [40 * i for i in range(13)]
NCH = len(SIZES)
NBUF = 3


def _make_embed_kernel(batch, seq):
    n_tokens = batch * seq
    assert n_tokens == NW * PER_W
    w_per_row = seq // PER_W

    mesh = plsc.VectorSubcoreMesh(core_axis_name="c", subcore_axis_name="s")

    @functools.partial(
        pl.kernel,
        out_type=jax.ShapeDtypeStruct((n_tokens, D_MODEL), jnp.float32),
        mesh=mesh,
        scratch_types=[
            pltpu.VMEM((PER_W,), jnp.int32),
            pltpu.VMEM((SIZES[0], D_MODEL), jnp.float32),
            pltpu.VMEM((SIZES[0], D_MODEL), jnp.float32),
            pltpu.VMEM((SIZES[0], D_MODEL), jnp.float32),
            pltpu.SemaphoreType.DMA,
            pltpu.SemaphoreType.DMA,
            pltpu.SemaphoreType.DMA,
            pltpu.SemaphoreType.DMA,
            pltpu.SemaphoreType.DMA,
            pltpu.SemaphoreType.DMA,
        ],
    )
    def embed(tokens_hbm, table_hbm, out_hbm, idx_v, rows_a, rows_b, rows_c,
              gsem0, gsem1, gsem2, wsem0, wsem1, wsem2):
        rows = (rows_a, rows_b, rows_c)
        gsem = (gsem0, gsem1, gsem2)
        wsem = (wsem0, wsem1, wsem2)
        wid = lax.axis_index("s") * NC + lax.axis_index("c")
        b = wid // w_per_row
        off = (wid % w_per_row) * PER_W
        pltpu.sync_copy(tokens_hbm.at[b, pl.ds(off, PER_W)], idx_v)
        base = wid * PER_W

        def gather(c):
            return pltpu.async_copy(
                table_hbm.at[idx_v.at[pl.ds(OFFS[c], SIZES[c])]],
                rows[c % NBUF].at[pl.ds(0, SIZES[c])], gsem[c % NBUF])

        def write(c):
            return pltpu.async_copy(
                rows[c % NBUF].at[pl.ds(0, SIZES[c])],
                out_hbm.at[pl.ds(base + OFFS[c], SIZES[c])], wsem[c % NBUF])

        gd = [None] * NCH
        wd = [None] * NCH
        gd[0] = gather(0)
        gd[1] = gather(1)
        for c in range(NCH):
            gd[c].wait()
            wd[c] = write(c)
            if c >= 1:
                wd[c - 1].wait()
            if c + 2 < NCH:
                gd[c + 2] = gather(c + 2)
        wd[NCH - 1].wait()

    return embed


@jax.jit
def kernel(tokens, W_E):
    batch, seq = tokens.shape
    out = _make_embed_kernel(batch, seq)(tokens.astype(jnp.int32), W_E)
    return out.reshape(batch, seq, D_MODEL)
